# Initial kernel scaffold; baseline (speedup 1.0000x reference)
import functools

import jax
import jax.numpy as jnp
import numpy as np
from jax import lax
from jax.experimental import pallas as pl
from jax.experimental.pallas import tpu as pltpu
from jax.experimental.pallas import tpu_sc as plsc

N = 2048
ROWS_PER_BLOCK = 128

_PERM = np.asarray(jax.random.permutation(jax.random.key(42), 48))
_IPERM = np.argsort(_PERM)


def _sort_block(v, ids):
    n, r = v.shape
    ii = lax.broadcasted_iota(jnp.int32, (n, 1), 0)
    k = 2
    while k <= n:
        j = k // 2
        while j >= 1:
            m = n // (2 * j)
            a = v.reshape(m, 2, j, r)
            vp = jnp.concatenate([a[:, 1:], a[:, :1]], axis=1).reshape(n, r)
            b = ids.reshape(m, 2, j, r)
            idp = jnp.concatenate([b[:, 1:], b[:, :1]], axis=1).reshape(n, r)
            ts = ((ii & k) == 0) == ((ii & j) == 0)
            less = (v < vp) | ((v == vp) & (ids < idp))
            take = less == ts
            v = jnp.where(take, v, vp)
            ids = jnp.where(take, ids, idp)
            j //= 2
        k *= 2
    return v, ids


def _sort_kernel_body(x_ref, bv_ref, idx_ref, nbv_ref):
    xb = x_ref[0]
    v = xb.T
    ids = lax.broadcasted_iota(jnp.int32, (N, ROWS_PER_BLOCK), 0)
    v, ids = _sort_block(v, ids)
    idx_ref[0] = ids.T
    nbv_ref[0] = 0.9 * bv_ref[0] + 0.1 * v.T


def _tc_sort(x3, bv3):
    B, C, _ = x3.shape
    cblocks = C // ROWS_PER_BLOCK
    iperm_t = jnp.asarray(_IPERM)

    grid = (B, cblocks)
    blk = (1, ROWS_PER_BLOCK, N)
    return pl.pallas_call(
        _sort_kernel_body,
        grid=grid,
        in_specs=[
            pl.BlockSpec(blk, lambda b, c: (b, c, 0)),
            pl.BlockSpec(blk, lambda b, c: (iperm_t[b], c, 0)),
        ],
        out_specs=[
            pl.BlockSpec(blk, lambda b, c: (b, c, 0)),
            pl.BlockSpec(blk, lambda b, c: (iperm_t[b], c, 0)),
        ],
        out_shape=[
            jax.ShapeDtypeStruct((B, C, N), jnp.int32),
            jax.ShapeDtypeStruct((B, C, N), jnp.float32),
        ],
        compiler_params=pltpu.CompilerParams(
            dimension_semantics=("arbitrary", "arbitrary"),
        ),
    )(x3, bv3)


def _make_sc_scatter(num_rows):
    mesh = plsc.VectorSubcoreMesh(core_axis_name="c", subcore_axis_name="s")
    info = plsc.get_sparse_core_info()
    nworkers = info.num_cores * info.num_subcores
    rows_per_worker = num_rows // nworkers
    assert num_rows % nworkers == 0

    @functools.partial(
        pl.kernel,
        mesh=mesh,
        out_type=jax.ShapeDtypeStruct((num_rows, N), jnp.float32),
        scratch_types=[
            pltpu.VMEM((N,), jnp.int32),
            pltpu.VMEM((N,), jnp.float32),
            pltpu.VMEM((N,), jnp.float32),
        ],
    )
    def scatter_kernel(idx_hbm, nbv_hbm, out_hbm, idx_v, nbv_v, out_v):
        wid = lax.axis_index("s") * info.num_cores + lax.axis_index("c")
        base = wid * rows_per_worker

        def body(i, carry):
            r = base + i
            pltpu.sync_copy(idx_hbm.at[r], idx_v)
            pltpu.sync_copy(nbv_hbm.at[r], nbv_v)
            for kk in range(N // 16):
                iv = idx_v[pl.ds(kk * 16, 16)]
                xv = nbv_v[pl.ds(kk * 16, 16)]
                plsc.store_scatter(out_v, [iv], xv)
            pltpu.sync_copy(out_v, out_hbm.at[r])
            return carry

        lax.fori_loop(0, rows_per_worker, body, 0)

    return scatter_kernel


def kernel(x, batch_value):
    B, C, W, H = x.shape
    x3 = x.reshape(B, C, N)
    idx, nbv = _tc_sort(x3, batch_value)
    sc = _make_sc_scatter(B * C)
    out = sc(idx.reshape(B * C, N), nbv.reshape(B * C, N))
    return out.reshape(B, C, W, H)


# speedup vs baseline: 1.8899x; 1.8899x over previous
import functools

import jax
import jax.numpy as jnp
import numpy as np
from jax import lax
from jax.experimental import pallas as pl
from jax.experimental.pallas import tpu as pltpu
from jax.experimental.pallas import tpu_sc as plsc

N = 2048
ROWS_PER_BLOCK = 128

def _inverse_batch_perm(B):
    perm = jax.random.permutation(jax.random.key(42), B)
    return jnp.argsort(perm).astype(jnp.int32)


def _stage(v_ref, id_ref, ii, j, k):
    n, r = v_ref.shape
    m = n // (2 * j)
    v = v_ref[...]
    ids = id_ref[...]
    a = v.reshape(m, 2, j, r)
    vp = jnp.concatenate([a[:, 1:], a[:, :1]], axis=1).reshape(n, r)
    b = ids.reshape(m, 2, j, r)
    idp = jnp.concatenate([b[:, 1:], b[:, :1]], axis=1).reshape(n, r)
    ts = ((ii & k) == 0) == ((ii & j) == 0)
    less = (v < vp) | ((v == vp) & (ids < idp))
    take = less == ts
    v_ref[...] = jnp.where(take, v, vp)
    id_ref[...] = jnp.where(take, ids, idp)


def _sort_kernel_body(iperm_ref, x_ref, bv_ref, idx_ref, nbv_ref,
                      v_ref, id_ref):
    del iperm_ref
    v_ref[...] = x_ref[0].T
    id_ref[...] = lax.broadcasted_iota(jnp.int32, (N, ROWS_PER_BLOCK), 0)
    ii = lax.broadcasted_iota(jnp.int32, (N, 1), 0)

    def phase(p, carry):
        k = jnp.left_shift(1, p)
        for q in range(10, -1, -1):
            @pl.when(q < p)
            def _():
                _stage(v_ref, id_ref, ii, 1 << q, k)
        return carry

    lax.fori_loop(1, 12, phase, 0)
    idx_ref[0] = id_ref[...].T
    nbv_ref[0] = 0.9 * bv_ref[0] + 0.1 * v_ref[...].T


def _tc_sort(x3, bv3):
    B, C, _ = x3.shape
    cblocks = C // ROWS_PER_BLOCK

    blk = (1, ROWS_PER_BLOCK, N)
    grid_spec = pltpu.PrefetchScalarGridSpec(
        num_scalar_prefetch=1,
        grid=(B, cblocks),
        in_specs=[
            pl.BlockSpec(blk, lambda b, c, ip: (b, c, 0)),
            pl.BlockSpec(blk, lambda b, c, ip: (ip[b], c, 0)),
        ],
        out_specs=[
            pl.BlockSpec(blk, lambda b, c, ip: (b, c, 0)),
            pl.BlockSpec(blk, lambda b, c, ip: (ip[b], c, 0)),
        ],
        scratch_shapes=[
            pltpu.VMEM((N, ROWS_PER_BLOCK), jnp.float32),
            pltpu.VMEM((N, ROWS_PER_BLOCK), jnp.int32),
        ],
    )
    return pl.pallas_call(
        _sort_kernel_body,
        grid_spec=grid_spec,
        out_shape=[
            jax.ShapeDtypeStruct((B, C, N), jnp.int32),
            jax.ShapeDtypeStruct((B, C, N), jnp.float32),
        ],
        compiler_params=pltpu.CompilerParams(
            dimension_semantics=("arbitrary", "arbitrary"),
        ),
    )(_inverse_batch_perm(B), x3, bv3)


def _make_sc_scatter(num_rows):
    mesh = plsc.VectorSubcoreMesh(core_axis_name="c", subcore_axis_name="s")
    info = plsc.get_sparse_core_info()
    nworkers = info.num_cores * info.num_subcores
    rows_per_worker = num_rows // nworkers
    assert num_rows % nworkers == 0

    @functools.partial(
        pl.kernel,
        mesh=mesh,
        out_type=jax.ShapeDtypeStruct((num_rows, N), jnp.float32),
        scratch_types=[
            pltpu.VMEM((N,), jnp.int32),
            pltpu.VMEM((N,), jnp.float32),
            pltpu.VMEM((N,), jnp.float32),
        ],
        compiler_params=pltpu.CompilerParams(needs_layout_passes=False),
    )
    def scatter_kernel(idx_hbm, nbv_hbm, out_hbm, idx_v, nbv_v, out_v):
        wid = lax.axis_index("s") * info.num_cores + lax.axis_index("c")
        base = wid * rows_per_worker

        def body(i, carry):
            r = base + i
            pltpu.sync_copy(idx_hbm.at[r], idx_v)
            pltpu.sync_copy(nbv_hbm.at[r], nbv_v)
            for kk in range(N // 16):
                iv = idx_v[pl.ds(kk * 16, 16)]
                xv = nbv_v[pl.ds(kk * 16, 16)]
                plsc.store_scatter(out_v, [iv], xv)
            pltpu.sync_copy(out_v, out_hbm.at[r])
            return carry

        lax.fori_loop(0, rows_per_worker, body, 0)

    return scatter_kernel


def kernel(x, batch_value):
    B, C, W, H = x.shape
    x3 = x.reshape(B, C, N)
    idx, nbv = _tc_sort(x3, batch_value)
    sc = _make_sc_scatter(B * C)
    out = sc(idx.reshape(B * C, N), nbv.reshape(B * C, N))
    return out.reshape(B, C, W, H)


# speedup vs baseline: 2.2839x; 1.2084x over previous
import functools

import jax
import jax.numpy as jnp
import numpy as np
from jax import lax
from jax.experimental import pallas as pl
from jax.experimental.pallas import tpu as pltpu
from jax.experimental.pallas import tpu_sc as plsc

N = 2048
ROWS_PER_BLOCK = 128

def _inverse_batch_perm(B):
    perm = jax.random.permutation(jax.random.key(42), B)
    return jnp.argsort(perm).astype(jnp.int32)


def _stage(v_ref, id_ref, ii, j, a_mask):
    n, r = v_ref.shape
    m = n // (2 * j)
    v = v_ref[...]
    ids = id_ref[...]
    a = v.reshape(m, 2, j, r)
    vp = jnp.concatenate([a[:, 1:], a[:, :1]], axis=1).reshape(n, r)
    b = ids.reshape(m, 2, j, r)
    idp = jnp.concatenate([b[:, 1:], b[:, :1]], axis=1).reshape(n, r)
    ab = a_mask ^ ((ii & j) == 0)
    less = (v < vp) | ((v == vp) & (ids < idp))
    take = less ^ ab
    v_ref[...] = jnp.where(take, vp, v)
    id_ref[...] = jnp.where(take, idp, ids)


def _sort_kernel_body(iperm_ref, x_ref, bv_ref, idx_ref, nbv_ref,
                      v_ref, id_ref):
    del iperm_ref
    v_ref[...] = x_ref[0].T
    id_ref[...] = lax.broadcasted_iota(jnp.int32, (N, ROWS_PER_BLOCK), 0)
    ii = lax.broadcasted_iota(jnp.int32, (N, 1), 0)

    def phase(p, carry):
        k = jnp.left_shift(1, p)
        a_mask = (ii & k) != 0
        for q in range(10, -1, -1):
            @pl.when(q < p)
            def _():
                _stage(v_ref, id_ref, ii, 1 << q, a_mask)
        return carry

    lax.fori_loop(1, 12, phase, 0)
    idx_ref[0] = id_ref[...].T
    nbv_ref[0] = 0.9 * bv_ref[0] + 0.1 * v_ref[...].T


def _tc_sort(x3, bv3):
    B, C, _ = x3.shape
    cblocks = C // ROWS_PER_BLOCK

    blk = (1, ROWS_PER_BLOCK, N)
    grid_spec = pltpu.PrefetchScalarGridSpec(
        num_scalar_prefetch=1,
        grid=(B, cblocks),
        in_specs=[
            pl.BlockSpec(blk, lambda b, c, ip: (b, c, 0)),
            pl.BlockSpec(blk, lambda b, c, ip: (ip[b], c, 0)),
        ],
        out_specs=[
            pl.BlockSpec(blk, lambda b, c, ip: (b, c, 0)),
            pl.BlockSpec(blk, lambda b, c, ip: (ip[b], c, 0)),
        ],
        scratch_shapes=[
            pltpu.VMEM((N, ROWS_PER_BLOCK), jnp.float32),
            pltpu.VMEM((N, ROWS_PER_BLOCK), jnp.int32),
        ],
    )
    return pl.pallas_call(
        _sort_kernel_body,
        grid_spec=grid_spec,
        out_shape=[
            jax.ShapeDtypeStruct((B, C, N), jnp.int32),
            jax.ShapeDtypeStruct((B, C, N), jnp.float32),
        ],
        compiler_params=pltpu.CompilerParams(
            dimension_semantics=("arbitrary", "arbitrary"),
        ),
    )(_inverse_batch_perm(B), x3, bv3)


_SC_ROWS = 8


def _make_sc_scatter(num_rows):
    mesh = plsc.VectorSubcoreMesh(core_axis_name="c", subcore_axis_name="s")
    info = plsc.get_sparse_core_info()
    nworkers = info.num_cores * info.num_subcores
    rows_per_worker = num_rows // nworkers
    groups = rows_per_worker // _SC_ROWS
    assert num_rows % (nworkers * _SC_ROWS) == 0

    @functools.partial(
        pl.kernel,
        mesh=mesh,
        out_type=jax.ShapeDtypeStruct((num_rows, N), jnp.float32),
        scratch_types=[
            pltpu.VMEM((_SC_ROWS, N), jnp.int32),
            pltpu.VMEM((_SC_ROWS, N), jnp.float32),
            pltpu.VMEM((_SC_ROWS, N), jnp.float32),
        ],
        compiler_params=pltpu.CompilerParams(needs_layout_passes=False),
    )
    def scatter_kernel(idx_hbm, nbv_hbm, out_hbm, idx_v, nbv_v, out_v):
        wid = lax.axis_index("s") * info.num_cores + lax.axis_index("c")
        base = wid * rows_per_worker
        rowvs = [jnp.full((16,), rr, jnp.int32) for rr in range(_SC_ROWS)]

        def body(g, carry):
            r = base + g * _SC_ROWS
            pltpu.sync_copy(idx_hbm.at[pl.ds(r, _SC_ROWS)], idx_v)
            pltpu.sync_copy(nbv_hbm.at[pl.ds(r, _SC_ROWS)], nbv_v)

            def chunk(kk, c2):
                for rr in range(_SC_ROWS):
                    iv = idx_v[rr, pl.ds(kk * 16, 16)]
                    xv = nbv_v[rr, pl.ds(kk * 16, 16)]
                    plsc.store_scatter(out_v, [rowvs[rr], iv], xv)
                return c2

            lax.fori_loop(0, N // 16, chunk, 0)
            pltpu.sync_copy(out_v, out_hbm.at[pl.ds(r, _SC_ROWS)])
            return carry

        lax.fori_loop(0, groups, body, 0)

    return scatter_kernel


def kernel(x, batch_value):
    B, C, W, H = x.shape
    x3 = x.reshape(B, C, N)
    idx, nbv = _tc_sort(x3, batch_value)
    sc = _make_sc_scatter(B * C)
    out = sc(idx.reshape(B * C, N), nbv.reshape(B * C, N))
    return out.reshape(B, C, W, H)


# speedup vs baseline: 2.3342x; 1.0220x over previous
import functools

import jax
import jax.numpy as jnp
import numpy as np
from jax import lax
from jax.experimental import pallas as pl
from jax.experimental.pallas import tpu as pltpu
from jax.experimental.pallas import tpu_sc as plsc

N = 2048
ROWS_PER_BLOCK = 128

def _inverse_batch_perm(B):
    perm = jax.random.permutation(jax.random.key(42), B)
    return jnp.argsort(perm).astype(jnp.int32)


def _stage(v_ref, id_ref, ii, j, a_mask):
    n, r = v_ref.shape
    m = n // (2 * j)
    v = v_ref[...]
    ids = id_ref[...]
    a = v.reshape(m, 2, j, r)
    vp = jnp.concatenate([a[:, 1:], a[:, :1]], axis=1).reshape(n, r)
    b = ids.reshape(m, 2, j, r)
    idp = jnp.concatenate([b[:, 1:], b[:, :1]], axis=1).reshape(n, r)
    ab = a_mask ^ ((ii & j) == 0)
    less = (v < vp) | ((v == vp) & (ids < idp))
    take = less ^ ab
    v_ref[...] = jnp.where(take, vp, v)
    id_ref[...] = jnp.where(take, idp, ids)


def _sort_kernel_body(iperm_ref, x_ref, bv_ref, idx_ref, nbv_ref,
                      v_ref, id_ref):
    del iperm_ref
    v_ref[...] = x_ref[0].T
    id_ref[...] = lax.broadcasted_iota(jnp.int32, (N, ROWS_PER_BLOCK), 0)
    ii = lax.broadcasted_iota(jnp.int32, (N, 1), 0)

    def phase(p, carry):
        k = jnp.left_shift(1, p)
        a_mask = (ii & k) != 0
        for q in range(10, -1, -1):
            @pl.when(q < p)
            def _():
                _stage(v_ref, id_ref, ii, 1 << q, a_mask)
        return carry

    lax.fori_loop(1, 12, phase, 0)
    idx_ref[0] = id_ref[...].T
    nbv_ref[0] = 0.9 * bv_ref[0] + 0.1 * v_ref[...].T


def _tc_sort(x3, bv3):
    B, C, _ = x3.shape
    cblocks = C // ROWS_PER_BLOCK

    blk = (1, ROWS_PER_BLOCK, N)
    grid_spec = pltpu.PrefetchScalarGridSpec(
        num_scalar_prefetch=1,
        grid=(B, cblocks),
        in_specs=[
            pl.BlockSpec(blk, lambda b, c, ip: (b, c, 0)),
            pl.BlockSpec(blk, lambda b, c, ip: (ip[b], c, 0)),
        ],
        out_specs=[
            pl.BlockSpec(blk, lambda b, c, ip: (b, c, 0)),
            pl.BlockSpec(blk, lambda b, c, ip: (ip[b], c, 0)),
        ],
        scratch_shapes=[
            pltpu.VMEM((N, ROWS_PER_BLOCK), jnp.float32),
            pltpu.VMEM((N, ROWS_PER_BLOCK), jnp.int32),
        ],
    )
    return pl.pallas_call(
        _sort_kernel_body,
        grid_spec=grid_spec,
        out_shape=[
            jax.ShapeDtypeStruct((B, C, N), jnp.int32),
            jax.ShapeDtypeStruct((B, C, N), jnp.float32),
        ],
        compiler_params=pltpu.CompilerParams(
            dimension_semantics=("arbitrary", "arbitrary"),
        ),
    )(_inverse_batch_perm(B), x3, bv3)


_SC_ROWS = 8


def _make_sc_scatter(num_rows):
    mesh = plsc.VectorSubcoreMesh(core_axis_name="c", subcore_axis_name="s")
    info = plsc.get_sparse_core_info()
    nworkers = info.num_cores * info.num_subcores
    rows_per_worker = num_rows // nworkers
    groups = rows_per_worker // _SC_ROWS
    assert num_rows % (nworkers * _SC_ROWS) == 0

    @functools.partial(
        pl.kernel,
        mesh=mesh,
        out_type=jax.ShapeDtypeStruct((num_rows, N), jnp.float32),
        scratch_types=[
            pltpu.VMEM((_SC_ROWS, N), jnp.int32),
            pltpu.VMEM((_SC_ROWS, N), jnp.float32),
            pltpu.VMEM((_SC_ROWS, N), jnp.float32),
        ],
        compiler_params=pltpu.CompilerParams(needs_layout_passes=False),
    )
    def scatter_kernel(idx_hbm, nbv_hbm, out_hbm, idx_v, nbv_v, out_v):
        wid = lax.axis_index("s") * info.num_cores + lax.axis_index("c")
        base = wid * rows_per_worker
        rowvs = [jnp.full((16,), rr, jnp.int32) for rr in range(_SC_ROWS)]

        def body(g, carry):
            r = base + g * _SC_ROWS
            pltpu.sync_copy(idx_hbm.at[pl.ds(r, _SC_ROWS)], idx_v)
            pltpu.sync_copy(nbv_hbm.at[pl.ds(r, _SC_ROWS)], nbv_v)

            def chunk(kk, c2):
                for rr in range(_SC_ROWS):
                    iv = idx_v[rr, pl.ds(kk * 16, 16)]
                    xv = nbv_v[rr, pl.ds(kk * 16, 16)]
                    plsc.store_scatter(out_v, [rowvs[rr], iv], xv)
                return c2

            lax.fori_loop(0, N // 16, chunk, 0)
            pltpu.sync_copy(out_v, out_hbm.at[pl.ds(r, _SC_ROWS)])
            return carry

        lax.fori_loop(0, groups, body, 0)

    return scatter_kernel


def kernel(x, batch_value):
    B, C, W, H = x.shape
    x3 = x.reshape(B, C, N)
    half = C // 2
    sc = _make_sc_scatter(B * half)
    outs = []
    for s in range(2):
        xh = x3[:, s * half:(s + 1) * half]
        bvh = batch_value[:, s * half:(s + 1) * half]
        idx, nbv = _tc_sort(xh, bvh)
        outs.append(sc(idx.reshape(B * half, N), nbv.reshape(B * half, N))
                    .reshape(B, half, W, H))
    return jnp.concatenate(outs, axis=1)


# speedup vs baseline: 2.6612x; 1.1401x over previous
import functools

import jax
import jax.numpy as jnp
import numpy as np
from jax import lax
from jax.experimental import pallas as pl
from jax.experimental.pallas import tpu as pltpu
from jax.experimental.pallas import tpu_sc as plsc

N = 2048
ROWS_PER_BLOCK = 256

def _inverse_batch_perm(B):
    perm = jax.random.permutation(jax.random.key(42), B)
    return jnp.argsort(perm).astype(jnp.int32)


def _stage(v_ref, id_ref, ii, j, a_mask):
    n, r = v_ref.shape
    m = n // (2 * j)
    v = v_ref[...]
    ids = id_ref[...]
    a = v.reshape(m, 2, j, r)
    vp = jnp.concatenate([a[:, 1:], a[:, :1]], axis=1).reshape(n, r)
    b = ids.reshape(m, 2, j, r)
    idp = jnp.concatenate([b[:, 1:], b[:, :1]], axis=1).reshape(n, r)
    ab = a_mask ^ ((ii & j) == 0)
    less = (v < vp) | ((v == vp) & (ids < idp))
    take = less ^ ab
    v_ref[...] = jnp.where(take, vp, v)
    id_ref[...] = jnp.where(take, idp, ids)


def _sort_kernel_body(iperm_ref, x_ref, bv_ref, idx_ref, nbv_ref,
                      v_ref, id_ref):
    del iperm_ref
    v_ref[...] = x_ref[0].T
    id_ref[...] = lax.broadcasted_iota(jnp.int32, (N, ROWS_PER_BLOCK), 0)
    ii = lax.broadcasted_iota(jnp.int32, (N, 1), 0)

    def phase(p, carry):
        k = jnp.left_shift(1, p)
        a_mask = (ii & k) != 0
        for q in range(10, -1, -1):
            @pl.when(q < p)
            def _():
                _stage(v_ref, id_ref, ii, 1 << q, a_mask)
        return carry

    lax.fori_loop(1, 12, phase, 0)
    idx_ref[0] = id_ref[...].T
    nbv_ref[0] = 0.9 * bv_ref[0] + 0.1 * v_ref[...].T


def _tc_sort(x3, bv3):
    B, C, _ = x3.shape
    cblocks = C // ROWS_PER_BLOCK

    blk = (1, ROWS_PER_BLOCK, N)
    grid_spec = pltpu.PrefetchScalarGridSpec(
        num_scalar_prefetch=1,
        grid=(B, cblocks),
        in_specs=[
            pl.BlockSpec(blk, lambda b, c, ip: (b, c, 0)),
            pl.BlockSpec(blk, lambda b, c, ip: (ip[b], c, 0)),
        ],
        out_specs=[
            pl.BlockSpec(blk, lambda b, c, ip: (b, c, 0)),
            pl.BlockSpec(blk, lambda b, c, ip: (ip[b], c, 0)),
        ],
        scratch_shapes=[
            pltpu.VMEM((N, ROWS_PER_BLOCK), jnp.float32),
            pltpu.VMEM((N, ROWS_PER_BLOCK), jnp.int32),
        ],
    )
    return pl.pallas_call(
        _sort_kernel_body,
        grid_spec=grid_spec,
        out_shape=[
            jax.ShapeDtypeStruct((B, C, N), jnp.int32),
            jax.ShapeDtypeStruct((B, C, N), jnp.float32),
        ],
        compiler_params=pltpu.CompilerParams(
            dimension_semantics=("arbitrary", "arbitrary"),
        ),
    )(_inverse_batch_perm(B), x3, bv3)


_SC_ROWS = 8


def _make_sc_scatter(num_rows):
    mesh = plsc.VectorSubcoreMesh(core_axis_name="c", subcore_axis_name="s")
    info = plsc.get_sparse_core_info()
    nworkers = info.num_cores * info.num_subcores
    rows_per_worker = num_rows // nworkers
    groups = rows_per_worker // _SC_ROWS
    assert num_rows % (nworkers * _SC_ROWS) == 0

    @functools.partial(
        pl.kernel,
        mesh=mesh,
        out_type=jax.ShapeDtypeStruct((num_rows, N), jnp.float32),
        scratch_types=[
            pltpu.VMEM((_SC_ROWS, N), jnp.int32),
            pltpu.VMEM((_SC_ROWS, N), jnp.float32),
            pltpu.VMEM((_SC_ROWS, N), jnp.float32),
        ],
        compiler_params=pltpu.CompilerParams(needs_layout_passes=False),
    )
    def scatter_kernel(idx_hbm, nbv_hbm, out_hbm, idx_v, nbv_v, out_v):
        wid = lax.axis_index("s") * info.num_cores + lax.axis_index("c")
        base = wid * rows_per_worker
        rowvs = [jnp.full((16,), rr, jnp.int32) for rr in range(_SC_ROWS)]

        def body(g, carry):
            r = base + g * _SC_ROWS
            pltpu.sync_copy(idx_hbm.at[pl.ds(r, _SC_ROWS)], idx_v)
            pltpu.sync_copy(nbv_hbm.at[pl.ds(r, _SC_ROWS)], nbv_v)

            def chunk(kk, c2):
                for rr in range(_SC_ROWS):
                    iv = idx_v[rr, pl.ds(kk * 16, 16)]
                    xv = nbv_v[rr, pl.ds(kk * 16, 16)]
                    plsc.store_scatter(out_v, [rowvs[rr], iv], xv)
                return c2

            lax.fori_loop(0, N // 16, chunk, 0)
            pltpu.sync_copy(out_v, out_hbm.at[pl.ds(r, _SC_ROWS)])
            return carry

        lax.fori_loop(0, groups, body, 0)

    return scatter_kernel


def kernel(x, batch_value):
    B, C, W, H = x.shape
    x3 = x.reshape(B, C, N)
    nsplit = 2 if C // 2 >= ROWS_PER_BLOCK else 1
    half = C // nsplit
    sc = _make_sc_scatter(B * half)
    outs = []
    for s in range(nsplit):
        xh = x3[:, s * half:(s + 1) * half]
        bvh = batch_value[:, s * half:(s + 1) * half]
        idx, nbv = _tc_sort(xh, bvh)
        outs.append(sc(idx.reshape(B * half, N), nbv.reshape(B * half, N))
                    .reshape(B, half, W, H))
    return outs[0] if nsplit == 1 else jnp.concatenate(outs, axis=1)


# speedup vs baseline: 2.6773x; 1.0060x over previous
import functools

import jax
import jax.numpy as jnp
import numpy as np
from jax import lax
from jax.experimental import pallas as pl
from jax.experimental.pallas import tpu as pltpu
from jax.experimental.pallas import tpu_sc as plsc

N = 2048
ROWS_PER_BLOCK = 256

def _inverse_batch_perm(B):
    perm = jax.random.permutation(jax.random.key(42), B)
    return jnp.argsort(perm).astype(jnp.int32)


def _stage(v_ref, id_ref, ii, j, a_mask):
    n, r = v_ref.shape
    m = n // (2 * j)
    v = v_ref[...]
    ids = id_ref[...]
    a = v.reshape(m, 2, j, r)
    vp = jnp.concatenate([a[:, 1:], a[:, :1]], axis=1).reshape(n, r)
    b = ids.reshape(m, 2, j, r)
    idp = jnp.concatenate([b[:, 1:], b[:, :1]], axis=1).reshape(n, r)
    ab = a_mask ^ ((ii & j) == 0)
    less = (v < vp) | ((v == vp) & (ids < idp))
    take = less ^ ab
    v_ref[...] = jnp.where(take, vp, v)
    id_ref[...] = jnp.where(take, idp, ids)


def _sort_kernel_body(iperm_ref, x_ref, bv_ref, idx_ref, nbv_ref,
                      v_ref, id_ref):
    del iperm_ref
    v_ref[...] = x_ref[0].T
    id_ref[...] = lax.broadcasted_iota(jnp.int32, (N, ROWS_PER_BLOCK), 0)
    ii = lax.broadcasted_iota(jnp.int32, (N, 1), 0)

    def phase(p, carry):
        k = jnp.left_shift(1, p)
        a_mask = (ii & k) != 0
        for q in range(10, -1, -1):
            @pl.when(q < p)
            def _():
                _stage(v_ref, id_ref, ii, 1 << q, a_mask)
        return carry

    lax.fori_loop(1, 12, phase, 0)
    idx_ref[0] = id_ref[...].T
    nbv_ref[0] = 0.9 * bv_ref[0] + 0.1 * v_ref[...].T


def _tc_sort(x3, bv3):
    B, C, _ = x3.shape
    cblocks = C // ROWS_PER_BLOCK

    blk = (1, ROWS_PER_BLOCK, N)
    grid_spec = pltpu.PrefetchScalarGridSpec(
        num_scalar_prefetch=1,
        grid=(B, cblocks),
        in_specs=[
            pl.BlockSpec(blk, lambda b, c, ip: (b, c, 0)),
            pl.BlockSpec(blk, lambda b, c, ip: (ip[b], c, 0)),
        ],
        out_specs=[
            pl.BlockSpec(blk, lambda b, c, ip: (b, c, 0)),
            pl.BlockSpec(blk, lambda b, c, ip: (ip[b], c, 0)),
        ],
        scratch_shapes=[
            pltpu.VMEM((N, ROWS_PER_BLOCK), jnp.float32),
            pltpu.VMEM((N, ROWS_PER_BLOCK), jnp.int32),
        ],
    )
    return pl.pallas_call(
        _sort_kernel_body,
        grid_spec=grid_spec,
        out_shape=[
            jax.ShapeDtypeStruct((B, C, N), jnp.int32),
            jax.ShapeDtypeStruct((B, C, N), jnp.float32),
        ],
        compiler_params=pltpu.CompilerParams(
            dimension_semantics=("arbitrary", "arbitrary"),
        ),
    )(_inverse_batch_perm(B), x3, bv3)


_SC_ROWS = 16


def _make_sc_scatter(num_rows):
    mesh = plsc.VectorSubcoreMesh(core_axis_name="c", subcore_axis_name="s")
    info = plsc.get_sparse_core_info()
    nworkers = info.num_cores * info.num_subcores
    rows_per_worker = num_rows // nworkers
    groups = rows_per_worker // _SC_ROWS
    assert num_rows % (nworkers * _SC_ROWS) == 0

    @functools.partial(
        pl.kernel,
        mesh=mesh,
        out_type=jax.ShapeDtypeStruct((num_rows, N), jnp.float32),
        scratch_types=[
            pltpu.VMEM((_SC_ROWS, N), jnp.int32),
            pltpu.VMEM((_SC_ROWS, N), jnp.float32),
            pltpu.VMEM((_SC_ROWS, N), jnp.float32),
        ],
        compiler_params=pltpu.CompilerParams(needs_layout_passes=False),
    )
    def scatter_kernel(idx_hbm, nbv_hbm, out_hbm, idx_v, nbv_v, out_v):
        wid = lax.axis_index("s") * info.num_cores + lax.axis_index("c")
        base = wid * rows_per_worker
        rowvs = [jnp.full((16,), rr, jnp.int32) for rr in range(_SC_ROWS)]

        def body(g, carry):
            r = base + g * _SC_ROWS
            pltpu.sync_copy(idx_hbm.at[pl.ds(r, _SC_ROWS)], idx_v)
            pltpu.sync_copy(nbv_hbm.at[pl.ds(r, _SC_ROWS)], nbv_v)

            def chunk(kk, c2):
                for rr in range(_SC_ROWS):
                    iv = idx_v[rr, pl.ds(kk * 16, 16)]
                    xv = nbv_v[rr, pl.ds(kk * 16, 16)]
                    plsc.store_scatter(out_v, [rowvs[rr], iv], xv)
                return c2

            lax.fori_loop(0, N // 16, chunk, 0)
            pltpu.sync_copy(out_v, out_hbm.at[pl.ds(r, _SC_ROWS)])
            return carry

        lax.fori_loop(0, groups, body, 0)

    return scatter_kernel


def kernel(x, batch_value):
    B, C, W, H = x.shape
    x3 = x.reshape(B, C, N)
    nsplit = 2 if C // 2 >= ROWS_PER_BLOCK else 1
    half = C // nsplit
    sc = _make_sc_scatter(B * half)
    outs = []
    for s in range(nsplit):
        xh = x3[:, s * half:(s + 1) * half]
        bvh = batch_value[:, s * half:(s + 1) * half]
        idx, nbv = _tc_sort(xh, bvh)
        outs.append(sc(idx.reshape(B * half, N), nbv.reshape(B * half, N))
                    .reshape(B, half, W, H))
    return outs[0] if nsplit == 1 else jnp.concatenate(outs, axis=1)


# speedup vs baseline: 2.6775x; 1.0001x over previous
import functools

import jax
import jax.numpy as jnp
from jax import lax
from jax.experimental import pallas as pl
from jax.experimental.pallas import tpu as pltpu
from jax.experimental.pallas import tpu_sc as plsc

N = 2048
ROWS_PER_BLOCK = 256

def _inverse_batch_perm(B):
    perm = jax.random.permutation(jax.random.key(42), B)
    return jnp.argsort(perm).astype(jnp.int32)


def _stage(v_ref, id_ref, ii, j, a_mask):
    n, r = v_ref.shape
    m = n // (2 * j)
    v = v_ref[...]
    ids = id_ref[...]
    a = v.reshape(m, 2, j, r)
    vp = jnp.concatenate([a[:, 1:], a[:, :1]], axis=1).reshape(n, r)
    b = ids.reshape(m, 2, j, r)
    idp = jnp.concatenate([b[:, 1:], b[:, :1]], axis=1).reshape(n, r)
    ab = a_mask ^ ((ii & j) == 0)
    less = (v < vp) | ((v == vp) & (ids < idp))
    take = less ^ ab
    v_ref[...] = jnp.where(take, vp, v)
    id_ref[...] = jnp.where(take, idp, ids)


def _sort_kernel_body(iperm_ref, x_ref, bv_ref, idx_ref, nbv_ref,
                      v_ref, id_ref):
    del iperm_ref
    v_ref[...] = x_ref[0].T
    id_ref[...] = lax.broadcasted_iota(jnp.int32, (N, ROWS_PER_BLOCK), 0)
    ii = lax.broadcasted_iota(jnp.int32, (N, 1), 0)

    def phase(p, carry):
        k = jnp.left_shift(1, p)
        a_mask = (ii & k) != 0
        for q in range(10, -1, -1):
            @pl.when(q < p)
            def _():
                _stage(v_ref, id_ref, ii, 1 << q, a_mask)
        return carry

    lax.fori_loop(1, 12, phase, 0)
    idx_ref[0] = id_ref[...].T
    nbv_ref[0] = 0.9 * bv_ref[0] + 0.1 * v_ref[...].T


def _tc_sort(x3, bv3):
    B, C, _ = x3.shape
    cblocks = C // ROWS_PER_BLOCK

    blk = (1, ROWS_PER_BLOCK, N)
    grid_spec = pltpu.PrefetchScalarGridSpec(
        num_scalar_prefetch=1,
        grid=(B, cblocks),
        in_specs=[
            pl.BlockSpec(blk, lambda b, c, ip: (b, c, 0)),
            pl.BlockSpec(blk, lambda b, c, ip: (ip[b], c, 0)),
        ],
        out_specs=[
            pl.BlockSpec(blk, lambda b, c, ip: (b, c, 0)),
            pl.BlockSpec(blk, lambda b, c, ip: (ip[b], c, 0)),
        ],
        scratch_shapes=[
            pltpu.VMEM((N, ROWS_PER_BLOCK), jnp.float32),
            pltpu.VMEM((N, ROWS_PER_BLOCK), jnp.int32),
        ],
    )
    return pl.pallas_call(
        _sort_kernel_body,
        grid_spec=grid_spec,
        out_shape=[
            jax.ShapeDtypeStruct((B, C, N), jnp.int32),
            jax.ShapeDtypeStruct((B, C, N), jnp.float32),
        ],
        compiler_params=pltpu.CompilerParams(
            dimension_semantics=("arbitrary", "arbitrary"),
        ),
    )(_inverse_batch_perm(B), x3, bv3)


_SC_ROWS = 16


def _make_sc_scatter(num_rows):
    mesh = plsc.VectorSubcoreMesh(core_axis_name="c", subcore_axis_name="s")
    info = plsc.get_sparse_core_info()
    nworkers = info.num_cores * info.num_subcores
    rows_per_worker = num_rows // nworkers
    groups = rows_per_worker // _SC_ROWS
    assert num_rows % (nworkers * _SC_ROWS) == 0

    @functools.partial(
        pl.kernel,
        mesh=mesh,
        out_type=jax.ShapeDtypeStruct((num_rows, N), jnp.float32),
        scratch_types=[
            pltpu.VMEM((_SC_ROWS, N), jnp.int32),
            pltpu.VMEM((_SC_ROWS, N), jnp.float32),
            pltpu.VMEM((_SC_ROWS, N), jnp.float32),
        ],
        compiler_params=pltpu.CompilerParams(needs_layout_passes=False),
    )
    def scatter_kernel(idx_hbm, nbv_hbm, out_hbm, idx_v, nbv_v, out_v):
        wid = lax.axis_index("s") * info.num_cores + lax.axis_index("c")
        base = wid * rows_per_worker
        rowvs = [jnp.full((16,), rr, jnp.int32) for rr in range(_SC_ROWS)]

        def body(g, carry):
            r = base + g * _SC_ROWS
            pltpu.sync_copy(idx_hbm.at[pl.ds(r, _SC_ROWS)], idx_v)
            pltpu.sync_copy(nbv_hbm.at[pl.ds(r, _SC_ROWS)], nbv_v)

            def chunk(kk, c2):
                for rr in range(_SC_ROWS):
                    iv = idx_v[rr, pl.ds(kk * 16, 16)]
                    xv = nbv_v[rr, pl.ds(kk * 16, 16)]
                    plsc.store_scatter(out_v, [rowvs[rr], iv], xv)
                return c2

            lax.fori_loop(0, N // 16, chunk, 0)
            pltpu.sync_copy(out_v, out_hbm.at[pl.ds(r, _SC_ROWS)])
            return carry

        lax.fori_loop(0, groups, body, 0)

    return scatter_kernel


def kernel(x, batch_value):
    B, C, W, H = x.shape
    x3 = x.reshape(B, C, N)
    nsplit = 2 if C // 2 >= ROWS_PER_BLOCK else 1
    half = C // nsplit
    sc = _make_sc_scatter(B * half)
    outs = []
    for s in range(nsplit):
        xh = x3[:, s * half:(s + 1) * half]
        bvh = batch_value[:, s * half:(s + 1) * half]
        idx, nbv = _tc_sort(xh, bvh)
        outs.append(sc(idx.reshape(B * half, N), nbv.reshape(B * half, N))
                    .reshape(B, half, W, H))
    return outs[0] if nsplit == 1 else jnp.concatenate(outs, axis=1)
